# baseline (speedup 1.0000x reference)
import functools

import jax
import jax.numpy as jnp
from jax import lax
from jax.experimental import pallas as pl
from jax.experimental.pallas import tpu as pltpu
from jax.experimental.pallas import tpu_sc as plsc

B, S, D = 4, 2048, 768
P, L, K = 64, 5, 8
KL = K * L
LD = L * D
CHUNK = 1024
NCHUNK = S // CHUNK
EPS = 1e-12


def _routing_body(x_ref, pkey_ref, out_ref, sim_ref, idx_ref,
                  rsim_ref, acc_ref, copy_sem):
    i = pl.program_id(0)
    xb = x_ref[...]
    psum = jnp.sum(xb, axis=1)

    @pl.when(i == 0)
    def _():
        acc_ref[...] = psum

    @pl.when(i > 0)
    def _():
        acc_ref[...] = acc_ref[...] + psum

    cp = pltpu.make_async_copy(
        x_ref, out_ref.at[:, pl.ds(KL + i * CHUNK, CHUNK), :], copy_sem)
    cp.start()

    @pl.when(i == NCHUNK - 1)
    def _():
        mean = acc_ref[...] * (1.0 / S)
        xn = mean * jax.lax.rsqrt(
            jnp.maximum(jnp.sum(mean * mean, axis=1, keepdims=True), EPS))
        pk = pkey_ref[...]
        pn = pk * jax.lax.rsqrt(
            jnp.maximum(jnp.sum(pk * pk, axis=1, keepdims=True), EPS))
        sim = jax.lax.dot_general(
            xn, pn, (((1,), (1,)), ((), ())),
            preferred_element_type=jnp.float32,
            precision=jax.lax.Precision.HIGHEST)
        sim_ref[...] = sim

        lane = jax.lax.broadcasted_iota(jnp.int32, (B, P), 1)
        work = sim
        rs = jnp.float32(0.0)
        cols = []
        for k in range(K):
            m = jnp.max(work, axis=1, keepdims=True)
            sel = jnp.min(jnp.where(work == m, lane, P), axis=1,
                          keepdims=True)
            cols.append(sel)
            rs = rs + jnp.sum(m)
            work = jnp.where(lane == sel, -jnp.inf, work)
        idx_ref[...] = jnp.concatenate(cols, axis=1)
        rsim_ref[...] = (rs * (1.0 / B)).reshape(1, 1)

    cp.wait()


def _stage_a(x_embed, prompt_key):
    return pl.pallas_call(
        _routing_body,
        grid=(NCHUNK,),
        in_specs=[
            pl.BlockSpec((B, CHUNK, D), lambda i: (0, i, 0)),
            pl.BlockSpec((P, D), lambda i: (0, 0)),
        ],
        out_specs=[
            pl.BlockSpec(memory_space=pltpu.MemorySpace.HBM),
            pl.BlockSpec((B, P), lambda i: (0, 0)),
            pl.BlockSpec((B, K), lambda i: (0, 0)),
            pl.BlockSpec((1, 1), lambda i: (0, 0)),
        ],
        out_shape=[
            jax.ShapeDtypeStruct((B, KL + S, D), jnp.float32),
            jax.ShapeDtypeStruct((B, P), jnp.float32),
            jax.ShapeDtypeStruct((B, K), jnp.int32),
            jax.ShapeDtypeStruct((1, 1), jnp.float32),
        ],
        scratch_shapes=[
            pltpu.VMEM((B, D), jnp.float32),
            pltpu.SemaphoreType.DMA,
        ],
        compiler_params=pltpu.CompilerParams(
            dimension_semantics=("arbitrary",)),
    )(x_embed, prompt_key)


def _sc_gather_body(prompt_hbm, idx_hbm, bp_hbm, idx_v, rows_v, sem):
    wid = lax.axis_index("s") * 2 + lax.axis_index("c")

    @pl.when(wid < B)
    def _():
        pltpu.sync_copy(idx_hbm.at[wid], idx_v)
        pltpu.async_copy(prompt_hbm.at[idx_v], rows_v, sem).wait()
        for k in range(K):
            pltpu.sync_copy(rows_v.at[k],
                            bp_hbm.at[pl.ds(wid * KL * D + k * LD, LD)])


_sc_gather = functools.partial(
    pl.kernel,
    mesh=plsc.VectorSubcoreMesh(core_axis_name="c", subcore_axis_name="s"),
    out_type=jax.ShapeDtypeStruct((B * KL * D,), jnp.float32),
    scratch_types=[
        pltpu.VMEM((K,), jnp.int32),
        pltpu.VMEM((K, LD), jnp.float32),
        pltpu.SemaphoreType.DMA,
    ],
)(_sc_gather_body)


def _splice_body(outb_ref, bp_ref, out_ref, sem):
    cp = pltpu.make_async_copy(bp_ref, out_ref.at[:, pl.ds(0, KL), :], sem)
    cp.start()
    cp.wait()


def _stage_c(out_big, bp):
    return pl.pallas_call(
        _splice_body,
        in_specs=[
            pl.BlockSpec(memory_space=pltpu.MemorySpace.HBM),
            pl.BlockSpec((B, KL, D), lambda: (0, 0, 0)),
        ],
        out_specs=pl.BlockSpec(memory_space=pltpu.MemorySpace.HBM),
        out_shape=jax.ShapeDtypeStruct((B, KL + S, D), jnp.float32),
        scratch_shapes=[pltpu.SemaphoreType.DMA],
        input_output_aliases={0: 0},
    )(out_big, bp)


def kernel(x_embed, prompt, prompt_key):
    out_big, sim, idx, rsim = _stage_a(x_embed, prompt_key)
    bp = _sc_gather(prompt.reshape(P, LD), idx)
    out = _stage_c(out_big, bp.reshape(B, KL, D))
    return out, rsim.reshape(()), sim, idx


# speedup vs baseline: 1.6817x; 1.6817x over previous
import jax
import jax.numpy as jnp
from jax.experimental import pallas as pl
from jax.experimental.pallas import tpu as pltpu

B, S, D = 4, 2048, 768
P, L, K = 64, 5, 8
KL = K * L
CHUNK = 1024
NCHUNK = S // CHUNK
EPS = 1e-12


def _routing_body(x_ref, pkey_ref, out_ref, sim_ref, idx_ref,
                  rsim_ref, acc_ref, copy_sem):
    i = pl.program_id(0)
    xb = x_ref[...]
    psum = jnp.sum(xb, axis=1)

    @pl.when(i == 0)
    def _():
        acc_ref[...] = psum

    @pl.when(i > 0)
    def _():
        acc_ref[...] = acc_ref[...] + psum

    cp = pltpu.make_async_copy(
        x_ref, out_ref.at[:, pl.ds(KL + i * CHUNK, CHUNK), :], copy_sem)
    cp.start()

    @pl.when(i == NCHUNK - 1)
    def _():
        mean = acc_ref[...] * (1.0 / S)
        xn = mean * jax.lax.rsqrt(
            jnp.maximum(jnp.sum(mean * mean, axis=1, keepdims=True), EPS))
        pk = pkey_ref[...]
        pn = pk * jax.lax.rsqrt(
            jnp.maximum(jnp.sum(pk * pk, axis=1, keepdims=True), EPS))
        sim = jax.lax.dot_general(
            xn, pn, (((1,), (1,)), ((), ())),
            preferred_element_type=jnp.float32,
            precision=jax.lax.Precision.DEFAULT)
        sim_ref[...] = sim

        lane = jax.lax.broadcasted_iota(jnp.int32, (B, P), 1)
        work = sim
        rs = jnp.float32(0.0)
        cols = []
        for k in range(K):
            m = jnp.max(work, axis=1, keepdims=True)
            sel = jnp.min(jnp.where(work == m, lane, P), axis=1,
                          keepdims=True)
            cols.append(sel)
            rs = rs + jnp.sum(m)
            work = jnp.where(lane == sel, -jnp.inf, work)
        idx_ref[...] = jnp.concatenate(cols, axis=1)
        rsim_ref[...] = (rs * (1.0 / B)).reshape(1, 1)

    cp.wait()


def _stage_a(x_embed, prompt_key):
    return pl.pallas_call(
        _routing_body,
        grid=(NCHUNK,),
        in_specs=[
            pl.BlockSpec((B, CHUNK, D), lambda i: (0, i, 0)),
            pl.BlockSpec((P, D), lambda i: (0, 0)),
        ],
        out_specs=[
            pl.BlockSpec(memory_space=pltpu.MemorySpace.HBM),
            pl.BlockSpec((B, P), lambda i: (0, 0)),
            pl.BlockSpec((B, K), lambda i: (0, 0)),
            pl.BlockSpec((1, 1), lambda i: (0, 0)),
        ],
        out_shape=[
            jax.ShapeDtypeStruct((B, KL + S, D), jnp.float32),
            jax.ShapeDtypeStruct((B, P), jnp.float32),
            jax.ShapeDtypeStruct((B, K), jnp.int32),
            jax.ShapeDtypeStruct((1, 1), jnp.float32),
        ],
        scratch_shapes=[
            pltpu.VMEM((B, D), jnp.float32),
            pltpu.SemaphoreType.DMA,
        ],
        compiler_params=pltpu.CompilerParams(
            dimension_semantics=("arbitrary",)),
    )(x_embed, prompt_key)


def _gather_body(outb_ref, prompt_ref, idx_ref, out_ref, bp_ref, sem):
    idx = idx_ref[...]
    j320 = jax.lax.broadcasted_iota(jnp.int32, (B, L, P * L), 2)
    l320 = jax.lax.broadcasted_iota(jnp.int32, (B, L, P * L), 1)
    for k in range(K):
        sel = idx[:, k:k + 1]
        hk = ((j320 // L == sel[:, :, None]) &
              (j320 % L == l320)).astype(jnp.float32)
        bpk = jax.lax.dot_general(
            hk.reshape(B * L, P * L), prompt_ref[...],
            (((1,), (0,)), ((), ())),
            preferred_element_type=jnp.float32,
            precision=jax.lax.Precision.HIGHEST)
        bp_ref[:, pl.ds(k * L, L), :] = bpk.reshape(B, L, D)
    cp = pltpu.make_async_copy(bp_ref, out_ref.at[:, pl.ds(0, KL), :], sem)
    cp.start()
    cp.wait()


def _stage_c(out_big, prompt2, idx):
    return pl.pallas_call(
        _gather_body,
        in_specs=[
            pl.BlockSpec(memory_space=pltpu.MemorySpace.HBM),
            pl.BlockSpec((P * L, D), lambda: (0, 0)),
            pl.BlockSpec((B, K), lambda: (0, 0)),
        ],
        out_specs=pl.BlockSpec(memory_space=pltpu.MemorySpace.HBM),
        out_shape=jax.ShapeDtypeStruct((B, KL + S, D), jnp.float32),
        scratch_shapes=[
            pltpu.VMEM((B, KL, D), jnp.float32),
            pltpu.SemaphoreType.DMA,
        ],
        input_output_aliases={0: 0},
    )(out_big, prompt2, idx)


def kernel(x_embed, prompt, prompt_key):
    out_big, sim, idx, rsim = _stage_a(x_embed, prompt_key)
    out = _stage_c(out_big, prompt.reshape(P * L, D), idx)
    return out, rsim.reshape(()), sim, idx


# speedup vs baseline: 2.0782x; 1.2358x over previous
import jax
import jax.numpy as jnp
from jax.experimental import pallas as pl
from jax.experimental.pallas import tpu as pltpu

B, S, D = 4, 2048, 768
P, L, K = 64, 5, 8
KL = K * L
CHUNK = 1024
NCHUNK = S // CHUNK
EPS = 1e-12


def _routing_body(x_ref, pkey_ref, prompt_ref, out_ref, sim_ref, idx_ref,
                  rsim_ref, acc_ref, bp_ref, copy_sem, bp_sem):
    i = pl.program_id(0)
    xb = x_ref[...]
    psum = jnp.sum(xb, axis=1)

    @pl.when(i == 0)
    def _():
        acc_ref[...] = psum

    @pl.when(i > 0)
    def _():
        acc_ref[...] = acc_ref[...] + psum

    cp = pltpu.make_async_copy(
        x_ref, out_ref.at[:, pl.ds(KL + i * CHUNK, CHUNK), :], copy_sem)
    cp.start()

    @pl.when(i == NCHUNK - 1)
    def _():
        mean = acc_ref[...] * (1.0 / S)
        xn = mean * jax.lax.rsqrt(
            jnp.maximum(jnp.sum(mean * mean, axis=1, keepdims=True), EPS))
        pk = pkey_ref[...]
        pn = pk * jax.lax.rsqrt(
            jnp.maximum(jnp.sum(pk * pk, axis=1, keepdims=True), EPS))
        sim = jax.lax.dot_general(
            xn, pn, (((1,), (1,)), ((), ())),
            preferred_element_type=jnp.float32,
            precision=jax.lax.Precision.DEFAULT)
        sim_ref[...] = sim

        lane = jax.lax.broadcasted_iota(jnp.int32, (B, P), 1)
        j320 = jax.lax.broadcasted_iota(jnp.int32, (B, L, P * L), 2)
        l320 = jax.lax.broadcasted_iota(jnp.int32, (B, L, P * L), 1)
        work = sim
        rs = jnp.float32(0.0)
        cols = []
        for k in range(K):
            m = jnp.max(work, axis=1, keepdims=True)
            sel = jnp.min(jnp.where(work == m, lane, P), axis=1,
                          keepdims=True)
            cols.append(sel)
            rs = rs + jnp.sum(m)
            work = jnp.where(lane == sel, -jnp.inf, work)
            hk = ((j320 // L == sel[:, :, None]) &
                  (j320 % L == l320)).astype(jnp.float32)
            bpk = jax.lax.dot_general(
                hk.reshape(B * L, P * L), prompt_ref[...],
                (((1,), (0,)), ((), ())),
                preferred_element_type=jnp.float32,
                precision=jax.lax.Precision.HIGHEST)
            bp_ref[:, pl.ds(k * L, L), :] = bpk.reshape(B, L, D)
        idx_ref[...] = jnp.concatenate(cols, axis=1)
        rsim_ref[...] = (rs * (1.0 / B)).reshape(1, 1)
        bcp = pltpu.make_async_copy(
            bp_ref, out_ref.at[:, pl.ds(0, KL), :], bp_sem)
        bcp.start()
        bcp.wait()

    cp.wait()


def kernel(x_embed, prompt, prompt_key):
    prompt2 = prompt.reshape(P * L, D)
    out_big, sim, idx, rsim = pl.pallas_call(
        _routing_body,
        grid=(NCHUNK,),
        in_specs=[
            pl.BlockSpec((B, CHUNK, D), lambda i: (0, i, 0)),
            pl.BlockSpec((P, D), lambda i: (0, 0)),
            pl.BlockSpec((P * L, D), lambda i: (0, 0)),
        ],
        out_specs=[
            pl.BlockSpec(memory_space=pltpu.MemorySpace.HBM),
            pl.BlockSpec((B, P), lambda i: (0, 0)),
            pl.BlockSpec((B, K), lambda i: (0, 0)),
            pl.BlockSpec((1, 1), lambda i: (0, 0)),
        ],
        out_shape=[
            jax.ShapeDtypeStruct((B, KL + S, D), jnp.float32),
            jax.ShapeDtypeStruct((B, P), jnp.float32),
            jax.ShapeDtypeStruct((B, K), jnp.int32),
            jax.ShapeDtypeStruct((1, 1), jnp.float32),
        ],
        scratch_shapes=[
            pltpu.VMEM((B, D), jnp.float32),
            pltpu.VMEM((B, KL, D), jnp.float32),
            pltpu.SemaphoreType.DMA,
            pltpu.SemaphoreType.DMA,
        ],
        compiler_params=pltpu.CompilerParams(
            dimension_semantics=("arbitrary",)),
    )(x_embed, prompt_key, prompt2)
    return out_big, rsim.reshape(()), sim, idx
